# Initial kernel scaffold; baseline (speedup 1.0000x reference)
import jax
import jax.numpy as jnp
from jax.experimental import pallas as pl


def kernel(x, edge_index, edge_attr, batch, graph_features, W1, b1, W2, b2, Wfc, bfc):
    raise NotImplementedError("write your pallas kernel here")


# speedup vs baseline: 15.0456x; 15.0456x over previous
import functools

import jax
import jax.numpy as jnp
from jax import lax
from jax.experimental import pallas as pl
from jax.experimental.pallas import tpu as pltpu
from jax.experimental.pallas import tpu_sc as plsc

_N = 100000
_E = 6400000
_B = 64
_CH = 2000
_NW = 32
_NCHUNK = _E // _CH
_CPW = _NCHUNK // _NW
_ZCH = 2000
_RC = 2000

_mesh = plsc.VectorSubcoreMesh(core_axis_name="c", subcore_axis_name="s")


def _zero_vmem(ref, nwords):
    z16 = jnp.zeros((16,), jnp.float32)

    def body(i, _):
        ref[pl.ds(i * 16, 16)] = z16
        return 0

    lax.fori_loop(0, nwords // 16, body, 0)


def _deg_body(attr_hbm, dst_hbm, degp_hbm, ew_hbm,
              attr_v, dst_v, ew_v, zb_v, deg_sh, sem):
    cid = lax.axis_index("c")
    sid = lax.axis_index("s")
    wid = sid * 2 + cid

    _zero_vmem(zb_v, _ZCH)

    def zero_sh(k, _):
        c = sid + k * 16

        @pl.when(c < _N // _ZCH)
        def _():
            pltpu.sync_copy(zb_v, deg_sh.at[pl.ds(c * _ZCH, _ZCH)])

        return 0

    lax.fori_loop(0, 4, zero_sh, 0)
    plsc.subcore_barrier()

    base4 = lax.iota(jnp.int32, 16) * 4

    def chunk(k, _):
        c = wid + k * _NW
        pltpu.sync_copy(dst_hbm.at[pl.ds(c * _CH, _CH)], dst_v)
        pltpu.sync_copy(attr_hbm.at[pl.ds(c * _CH * 4, _CH * 4)], attr_v)

        def extract(j, _):
            ew_v[pl.ds(j * 16, 16)] = plsc.load_gather(attr_v, [base4 + j * 64])
            return 0

        lax.fori_loop(0, _CH // 16, extract, 0)
        pltpu.sync_copy(ew_v, ew_hbm.at[pl.ds(c * _CH, _CH)])
        pltpu.sync_copy(ew_v, deg_sh.at[dst_v], add=True)
        return 0

    lax.fori_loop(0, _CPW, chunk, 0)
    plsc.subcore_barrier()

    @pl.when(sid == 0)
    def _():
        pltpu.sync_copy(deg_sh, degp_hbm.at[cid])


_deg_call = pl.kernel(
    _deg_body,
    out_type=[jax.ShapeDtypeStruct((2, _N), jnp.float32),
              jax.ShapeDtypeStruct((_E,), jnp.float32)],
    mesh=_mesh,
    compiler_params=pltpu.CompilerParams(needs_layout_passes=False),
    scratch_types=[
        pltpu.VMEM((_CH * 4,), jnp.float32),
        pltpu.VMEM((_CH,), jnp.int32),
        pltpu.VMEM((_CH,), jnp.float32),
        pltpu.VMEM((_ZCH,), jnp.float32),
        pltpu.VMEM_SHARED((_N,), jnp.float32),
        pltpu.SemaphoreType.DMA,
    ],
)


def _make_conv(F):
    n_rc = _N // _RC

    def body(*refs):
        (src_hbm, dst_hbm, ew_hbm, yt_hbm, zpt_hbm,
         src_v, dst_v, ew_v, col_v, zb_v) = refs[:10]
        y_sh = refs[10:10 + F]
        z_sh = refs[10 + F:10 + 2 * F]
        sem = refs[10 + 2 * F]
        cid = lax.axis_index("c")
        sid = lax.axis_index("s")
        wid = sid * 2 + cid

        _zero_vmem(zb_v, _RC)

        def stage(k, _):
            c = sid + k * 16

            @pl.when(c < n_rc)
            def _():
                sl = pl.ds(c * _RC, _RC)
                for f in range(F):
                    pltpu.sync_copy(yt_hbm.at[pl.ds(f * _N + c * _RC, _RC)],
                                    y_sh[f].at[sl])
                    pltpu.sync_copy(zb_v, z_sh[f].at[sl])

            return 0

        lax.fori_loop(0, pl.cdiv(n_rc, 16), stage, 0)
        plsc.subcore_barrier()

        def chunk(k, _):
            c = wid + k * _NW
            pltpu.sync_copy(src_hbm.at[pl.ds(c * _CH, _CH)], src_v)
            pltpu.sync_copy(dst_hbm.at[pl.ds(c * _CH, _CH)], dst_v)
            pltpu.sync_copy(ew_hbm.at[pl.ds(c * _CH, _CH)], ew_v)
            for f in range(F):
                pltpu.async_copy(y_sh[f].at[src_v], col_v, sem).wait()

                def scale(j, _):
                    sl = pl.ds(j * 16, 16)
                    col_v[sl] = col_v[sl] * ew_v[sl]
                    return 0

                lax.fori_loop(0, _CH // 16, scale, 0)
                pltpu.sync_copy(col_v, z_sh[f].at[dst_v], add=True)
            return 0

        lax.fori_loop(0, _CPW, chunk, 0)
        plsc.subcore_barrier()

        def wb(k, _):
            c = sid + k * 16

            @pl.when(c < n_rc)
            def _():
                for f in range(F):
                    pltpu.sync_copy(
                        z_sh[f].at[pl.ds(c * _RC, _RC)],
                        zpt_hbm.at[pl.ds(cid * F * _N + f * _N + c * _RC, _RC)])

            return 0

        lax.fori_loop(0, pl.cdiv(n_rc, 16), wb, 0)

    return pl.kernel(
        body,
        out_type=jax.ShapeDtypeStruct((2 * F * _N,), jnp.float32),
        mesh=_mesh,
        compiler_params=pltpu.CompilerParams(
            needs_layout_passes=False, use_tc_tiling_on_sc=False),
        scratch_types=(
            [pltpu.VMEM((_CH,), jnp.int32),
             pltpu.VMEM((_CH,), jnp.int32),
             pltpu.VMEM((_CH,), jnp.float32),
             pltpu.VMEM((_CH,), jnp.float32),
             pltpu.VMEM((_RC,), jnp.float32)]
            + [pltpu.VMEM_SHARED((_N,), jnp.float32) for _ in range(2 * F)]
            + [pltpu.SemaphoreType.DMA]
        ),
    )


_conv4 = _make_conv(4)
_conv2 = _make_conv(2)


def _tc_prep_body(degp_r, xt_r, w1_r, dinv_o, y1t_o):
    deg = degp_r[0] + degp_r[1] + 1.0
    dinv = jnp.where(deg > 0, lax.rsqrt(deg), 0.0)
    xt = xt_r[...]
    w = w1_r[...]
    dinv_o[...] = dinv
    for j in range(4):
        acc = w[0, j] * xt[0] + w[1, j] * xt[1] + w[2, j] * xt[2] + w[3, j] * xt[3]
        y1t_o[j, :] = dinv * acc


_tc_prep = pl.pallas_call(
    _tc_prep_body,
    out_shape=(jax.ShapeDtypeStruct((_N,), jnp.float32),
               jax.ShapeDtypeStruct((4, _N), jnp.float32)),
)


def _tc_mid_body(zpt_r, y1t_r, dinv_r, w2_r, b1_r, y2t_o):
    dinv = dinv_r[...]
    w = w2_r[...]
    h = [None] * 4
    for k in range(4):
        zk = zpt_r[0, k] + zpt_r[1, k] + y1t_r[k]
        h[k] = jnp.maximum(dinv * zk + b1_r[k], 0.0)
    for j in range(2):
        acc = w[0, j] * h[0] + w[1, j] * h[1] + w[2, j] * h[2] + w[3, j] * h[3]
        y2t_o[j, :] = dinv * acc


_tc_mid = pl.pallas_call(
    _tc_mid_body,
    out_shape=jax.ShapeDtypeStruct((2, _N), jnp.float32),
)


def _tc_final_body(zpt_r, y2t_r, dinv_r, b2_r, batch_r, gf_r, wfc_r, bfc_r, out_o):
    dinv = dinv_r[...]
    h0 = jnp.maximum(dinv * (zpt_r[0, 0] + zpt_r[1, 0] + y2t_r[0]) + b2_r[0], 0.0)
    h1 = jnp.maximum(dinv * (zpt_r[0, 1] + zpt_r[1, 1] + y2t_r[1]) + b2_r[1], 0.0)

    rb = 12800
    gids = lax.broadcasted_iota(jnp.int32, (_B, 1), 0)
    cnt = jnp.zeros((_B,), jnp.float32)
    s0 = jnp.zeros((_B,), jnp.float32)
    s1 = jnp.zeros((_B,), jnp.float32)
    off = 0
    while off < _N:
        nb = min(rb, _N - off)
        mb = lax.slice(batch_r[...], (off,), (off + nb,))[None, :] == gids
        mf = mb.astype(jnp.float32)
        cnt = cnt + jnp.sum(mf, axis=1)
        s0 = s0 + jnp.sum(mf * lax.slice(h0, (off,), (off + nb,))[None, :], axis=1)
        s1 = s1 + jnp.sum(mf * lax.slice(h1, (off,), (off + nb,))[None, :], axis=1)
        off += nb

    denom = jnp.maximum(cnt, 1.0)
    pooled0 = s0 / denom
    pooled1 = s1 / denom
    gf = gf_r[...]
    wfc = wfc_r[...]
    out = (pooled0[:, None] * wfc[0:1, :] + pooled1[:, None] * wfc[1:2, :]
           + gf[:, 0:1] * wfc[2:3, :] + gf[:, 1:2] * wfc[3:4, :]
           + gf[:, 2:3] * wfc[4:5, :] + gf[:, 3:4] * wfc[5:6, :])
    out_o[...] = out + bfc_r[...][None, :]


_tc_final = pl.pallas_call(
    _tc_final_body,
    out_shape=jax.ShapeDtypeStruct((_B, 1), jnp.float32),
)


def kernel(x, edge_index, edge_attr, batch, graph_features, W1, b1, W2, b2, Wfc, bfc):
    src = edge_index[0]
    dst = edge_index[1]
    attr_flat = edge_attr.reshape(-1)

    degp, ew = _deg_call(attr_flat, dst)
    dinv, y1t = _tc_prep(degp, x.T, W1)
    z1pt = _conv4(src, dst, ew, y1t.reshape(-1)).reshape(2, 4, _N)
    y2t = _tc_mid(z1pt, y1t, dinv, W2, b1)
    z2pt = _conv2(src, dst, ew, y2t.reshape(-1)).reshape(2, 2, _N)
    return _tc_final(z2pt, y2t, dinv, b2, batch,
                     graph_features, Wfc, bfc)


# speedup vs baseline: 15.3593x; 1.0209x over previous
import functools

import jax
import jax.numpy as jnp
from jax import lax
from jax.experimental import pallas as pl
from jax.experimental.pallas import tpu as pltpu
from jax.experimental.pallas import tpu_sc as plsc

_N = 100000
_E = 6400000
_B = 64
_CH = 4000
_NW = 32
_NCHUNK = _E // _CH
_CPW = _NCHUNK // _NW
_ZCH = 2000
_RC = 2000

_mesh = plsc.VectorSubcoreMesh(core_axis_name="c", subcore_axis_name="s")


def _zero_vmem(ref, nwords):
    z16 = jnp.zeros((16,), jnp.float32)

    def body(i, _):
        ref[pl.ds(i * 16, 16)] = z16
        return 0

    lax.fori_loop(0, nwords // 16, body, 0)


def _deg_body(attr_hbm, dst_hbm, degp_hbm, ew_hbm,
              attr_v, dst_v, ew_v, zb_v, deg_sh, sem):
    cid = lax.axis_index("c")
    sid = lax.axis_index("s")
    wid = sid * 2 + cid

    _zero_vmem(zb_v, _ZCH)

    def zero_sh(k, _):
        c = sid + k * 16

        @pl.when(c < _N // _ZCH)
        def _():
            pltpu.sync_copy(zb_v, deg_sh.at[pl.ds(c * _ZCH, _ZCH)])

        return 0

    lax.fori_loop(0, 4, zero_sh, 0)
    plsc.subcore_barrier()

    base4 = lax.iota(jnp.int32, 16) * 4

    def chunk(k, _):
        c = wid + k * _NW
        pltpu.sync_copy(dst_hbm.at[pl.ds(c * _CH, _CH)], dst_v)
        pltpu.sync_copy(attr_hbm.at[pl.ds(c * _CH * 4, _CH * 4)], attr_v)

        def extract(j, _):
            ew_v[pl.ds(j * 16, 16)] = plsc.load_gather(attr_v, [base4 + j * 64])
            return 0

        lax.fori_loop(0, _CH // 16, extract, 0)
        pltpu.sync_copy(ew_v, ew_hbm.at[pl.ds(c * _CH, _CH)])
        pltpu.sync_copy(ew_v, deg_sh.at[dst_v], add=True)
        return 0

    lax.fori_loop(0, _CPW, chunk, 0)
    plsc.subcore_barrier()

    @pl.when(sid == 0)
    def _():
        pltpu.sync_copy(deg_sh, degp_hbm.at[cid])


_deg_call = pl.kernel(
    _deg_body,
    out_type=[jax.ShapeDtypeStruct((2, _N), jnp.float32),
              jax.ShapeDtypeStruct((_E,), jnp.float32)],
    mesh=_mesh,
    compiler_params=pltpu.CompilerParams(needs_layout_passes=False),
    scratch_types=[
        pltpu.VMEM((_CH * 4,), jnp.float32),
        pltpu.VMEM((_CH,), jnp.int32),
        pltpu.VMEM((_CH,), jnp.float32),
        pltpu.VMEM((_ZCH,), jnp.float32),
        pltpu.VMEM_SHARED((_N,), jnp.float32),
        pltpu.SemaphoreType.DMA,
    ],
)


def _make_conv(F):
    n_rc = _N // _RC

    def body(*refs):
        (src_hbm, dst_hbm, ew_hbm, yt_hbm, zpt_hbm,
         src_v, dst_v, ew_v, col_v, zb_v) = refs[:10]
        y_sh = refs[10:10 + F]
        z_sh = refs[10 + F:10 + 2 * F]
        sem = refs[10 + 2 * F]
        cid = lax.axis_index("c")
        sid = lax.axis_index("s")
        wid = sid * 2 + cid

        _zero_vmem(zb_v, _RC)

        def stage(k, _):
            c = sid + k * 16

            @pl.when(c < n_rc)
            def _():
                sl = pl.ds(c * _RC, _RC)
                for f in range(F):
                    pltpu.sync_copy(yt_hbm.at[pl.ds(f * _N + c * _RC, _RC)],
                                    y_sh[f].at[sl])
                    pltpu.sync_copy(zb_v, z_sh[f].at[sl])

            return 0

        lax.fori_loop(0, pl.cdiv(n_rc, 16), stage, 0)
        plsc.subcore_barrier()

        def chunk(k, _):
            c = wid + k * _NW
            pltpu.sync_copy(src_hbm.at[pl.ds(c * _CH, _CH)], src_v)
            pltpu.sync_copy(dst_hbm.at[pl.ds(c * _CH, _CH)], dst_v)
            pltpu.sync_copy(ew_hbm.at[pl.ds(c * _CH, _CH)], ew_v)
            for f in range(F):
                pltpu.async_copy(y_sh[f].at[src_v], col_v, sem).wait()

                def scale(j, _):
                    sl = pl.ds(j * 16, 16)
                    col_v[sl] = col_v[sl] * ew_v[sl]
                    return 0

                lax.fori_loop(0, _CH // 16, scale, 0)
                pltpu.sync_copy(col_v, z_sh[f].at[dst_v], add=True)
            return 0

        lax.fori_loop(0, _CPW, chunk, 0)
        plsc.subcore_barrier()

        def wb(k, _):
            c = sid + k * 16

            @pl.when(c < n_rc)
            def _():
                for f in range(F):
                    pltpu.sync_copy(
                        z_sh[f].at[pl.ds(c * _RC, _RC)],
                        zpt_hbm.at[pl.ds(cid * F * _N + f * _N + c * _RC, _RC)])

            return 0

        lax.fori_loop(0, pl.cdiv(n_rc, 16), wb, 0)

    return pl.kernel(
        body,
        out_type=jax.ShapeDtypeStruct((2 * F * _N,), jnp.float32),
        mesh=_mesh,
        compiler_params=pltpu.CompilerParams(
            needs_layout_passes=False, use_tc_tiling_on_sc=False),
        scratch_types=(
            [pltpu.VMEM((_CH,), jnp.int32),
             pltpu.VMEM((_CH,), jnp.int32),
             pltpu.VMEM((_CH,), jnp.float32),
             pltpu.VMEM((_CH,), jnp.float32),
             pltpu.VMEM((_RC,), jnp.float32)]
            + [pltpu.VMEM_SHARED((_N,), jnp.float32) for _ in range(2 * F)]
            + [pltpu.SemaphoreType.DMA]
        ),
    )


_conv4 = _make_conv(4)
_conv2 = _make_conv(2)


def _tc_prep_body(degp_r, xt_r, w1_r, dinv_o, y1t_o):
    deg = degp_r[0] + degp_r[1] + 1.0
    dinv = jnp.where(deg > 0, lax.rsqrt(deg), 0.0)
    xt = xt_r[...]
    w = w1_r[...]
    dinv_o[...] = dinv
    for j in range(4):
        acc = w[0, j] * xt[0] + w[1, j] * xt[1] + w[2, j] * xt[2] + w[3, j] * xt[3]
        y1t_o[j, :] = dinv * acc


_tc_prep = pl.pallas_call(
    _tc_prep_body,
    out_shape=(jax.ShapeDtypeStruct((_N,), jnp.float32),
               jax.ShapeDtypeStruct((4, _N), jnp.float32)),
)


def _tc_mid_body(zpt_r, y1t_r, dinv_r, w2_r, b1_r, y2t_o):
    dinv = dinv_r[...]
    w = w2_r[...]
    h = [None] * 4
    for k in range(4):
        zk = zpt_r[0, k] + zpt_r[1, k] + y1t_r[k]
        h[k] = jnp.maximum(dinv * zk + b1_r[k], 0.0)
    for j in range(2):
        acc = w[0, j] * h[0] + w[1, j] * h[1] + w[2, j] * h[2] + w[3, j] * h[3]
        y2t_o[j, :] = dinv * acc


_tc_mid = pl.pallas_call(
    _tc_mid_body,
    out_shape=jax.ShapeDtypeStruct((2, _N), jnp.float32),
)


def _tc_final_body(zpt_r, y2t_r, dinv_r, b2_r, batch_r, gf_r, wfc_r, bfc_r, out_o):
    dinv = dinv_r[...]
    h0 = jnp.maximum(dinv * (zpt_r[0, 0] + zpt_r[1, 0] + y2t_r[0]) + b2_r[0], 0.0)
    h1 = jnp.maximum(dinv * (zpt_r[0, 1] + zpt_r[1, 1] + y2t_r[1]) + b2_r[1], 0.0)

    rb = 12800
    gids = lax.broadcasted_iota(jnp.int32, (_B, 1), 0)
    cnt = jnp.zeros((_B,), jnp.float32)
    s0 = jnp.zeros((_B,), jnp.float32)
    s1 = jnp.zeros((_B,), jnp.float32)
    off = 0
    while off < _N:
        nb = min(rb, _N - off)
        mb = lax.slice(batch_r[...], (off,), (off + nb,))[None, :] == gids
        mf = mb.astype(jnp.float32)
        cnt = cnt + jnp.sum(mf, axis=1)
        s0 = s0 + jnp.sum(mf * lax.slice(h0, (off,), (off + nb,))[None, :], axis=1)
        s1 = s1 + jnp.sum(mf * lax.slice(h1, (off,), (off + nb,))[None, :], axis=1)
        off += nb

    denom = jnp.maximum(cnt, 1.0)
    pooled0 = s0 / denom
    pooled1 = s1 / denom
    gf = gf_r[...]
    wfc = wfc_r[...]
    out = (pooled0[:, None] * wfc[0:1, :] + pooled1[:, None] * wfc[1:2, :]
           + gf[:, 0:1] * wfc[2:3, :] + gf[:, 1:2] * wfc[3:4, :]
           + gf[:, 2:3] * wfc[4:5, :] + gf[:, 3:4] * wfc[5:6, :])
    out_o[...] = out + bfc_r[...][None, :]


_tc_final = pl.pallas_call(
    _tc_final_body,
    out_shape=jax.ShapeDtypeStruct((_B, 1), jnp.float32),
)


def kernel(x, edge_index, edge_attr, batch, graph_features, W1, b1, W2, b2, Wfc, bfc):
    src = edge_index[0]
    dst = edge_index[1]
    attr_flat = edge_attr.reshape(-1)

    degp, ew = _deg_call(attr_flat, dst)
    dinv, y1t = _tc_prep(degp, x.T, W1)
    z1pt = _conv4(src, dst, ew, y1t.reshape(-1)).reshape(2, 4, _N)
    y2t = _tc_mid(z1pt, y1t, dinv, W2, b1)
    z2pt = _conv2(src, dst, ew, y2t.reshape(-1)).reshape(2, 2, _N)
    return _tc_final(z2pt, y2t, dinv, b2, batch,
                     graph_features, Wfc, bfc)
